# baseline (speedup 1.0000x reference)
import functools

import jax
import jax.numpy as jnp
from jax import lax
from jax.experimental import pallas as pl
from jax.experimental.pallas import tpu as pltpu
from jax.experimental.pallas import tpu_sc as plsc

NC = 2
NS = 16
NW = NC * NS
CH = 128


def _mesh():
    return plsc.VectorSubcoreMesh(
        core_axis_name="c", subcore_axis_name="s",
        num_cores=NC, num_subcores=NS)


@functools.lru_cache(maxsize=None)
def _deg_kernel(E, N):
    n_chunks = E // CH
    base_iters = n_chunks // NW
    extra = n_chunks % NW
    zlen = (N // NS) // 8 * 8
    rem = N - zlen * NS

    @functools.partial(
        pl.kernel,
        out_type=jax.ShapeDtypeStruct((NC * N,), jnp.float32),
        mesh=_mesh(),
        compiler_params=pltpu.CompilerParams(needs_layout_passes=False),
        scratch_types=[
            pltpu.VMEM_SHARED((NS * N,), jnp.float32),
            pltpu.VMEM((N,), jnp.float32),
            pltpu.VMEM((CH,), jnp.int32),
            pltpu.VMEM((CH,), jnp.int32),
            pltpu.VMEM((zlen,), jnp.float32),
            pltpu.VMEM((zlen,), jnp.float32),
            pltpu.SemaphoreType.DMA,
            pltpu.SemaphoreType.DMA,
        ],
    )
    def deg_k(dst_hbm, degp_hbm, stage, hist, didx0, didx1, abuf, sbuf,
              isem0, isem1):
        cid = lax.axis_index("c")
        sid = lax.axis_index("s")
        w = sid * NC + cid

        c0 = w * base_iters + jnp.minimum(w, extra)
        didx = (didx0, didx1)
        isem = (isem0, isem1)

        pltpu.async_copy(dst_hbm.at[c0, 0], didx0, isem0)
        pltpu.async_copy(dst_hbm.at[c0 + 1, 0], didx1, isem1)

        one16 = jnp.full((16,), 1.0, jnp.float32)
        z16 = jnp.zeros((16,), jnp.float32)

        def zfill(r, carry):
            hist[pl.ds(r * 16, 16)] = z16
            return carry
        lax.fori_loop(0, N // 16, zfill, 0)

        def body(r, b, idx_next):
            pltpu.make_async_copy(dst_hbm.at[c0, 0], didx[b],
                                  isem[b]).wait()
            for i in range(CH // 16):
                v = didx[b][pl.ds(i * 16, 16)]
                plsc.addupdate_scatter(hist, [v], one16)
            if idx_next:
                pltpu.async_copy(dst_hbm.at[c0 + r + 2, 0], didx[b], isem[b])

        def loop_body(i, carry):
            r = 2 * i
            body(r, 0, True)
            body(r + 1, 1, True)
            return carry
        lax.fori_loop(0, (base_iters - 2) // 2, loop_body, 0)

        body(base_iters - 2, 0, False)
        body(base_iters - 1, 1, False)

        @pl.when(w < extra)
        def _():
            pltpu.sync_copy(dst_hbm.at[c0 + base_iters, 0], didx0)
            for i in range(CH // 16):
                v = didx0[pl.ds(i * 16, 16)]
                plsc.addupdate_scatter(hist, [v], one16)

        pltpu.sync_copy(hist, stage.at[pl.ds(sid * N, N)])
        plsc.subcore_barrier()

        r0 = sid * zlen

        def czero(r, carry):
            abuf[pl.ds(r * 16, 16)] = z16
            return carry
        lax.fori_loop(0, zlen // 16, czero, 0)

        for j in range(NS):
            pltpu.sync_copy(stage.at[pl.ds(j * N + r0, zlen)], sbuf)

            def cadd(r, carry):
                s = pl.ds(r * 16, 16)
                abuf[s] = abuf[s] + sbuf[s]
                return carry
            lax.fori_loop(0, zlen // 16, cadd, 0)

        pltpu.sync_copy(abuf, degp_hbm.at[pl.ds(cid * N + r0, zlen)])

        @pl.when(sid == 0)
        def _():
            def tzero(r, carry):
                abuf[pl.ds(r * 16, 16)] = z16
                return carry
            lax.fori_loop(0, rem // 16, tzero, 0)
            for j in range(NS):
                pltpu.sync_copy(stage.at[pl.ds(j * N + NS * zlen, rem)],
                                sbuf.at[pl.ds(0, rem)])

                def tadd(r, carry):
                    s = pl.ds(r * 16, 16)
                    abuf[s] = abuf[s] + sbuf[s]
                    return carry
                lax.fori_loop(0, rem // 16, tadd, 0)
            pltpu.sync_copy(abuf.at[pl.ds(0, rem)],
                            degp_hbm.at[pl.ds(cid * N + NS * zlen, rem)])

    return deg_k


@functools.lru_cache(maxsize=None)
def _prop_kernel(E, N, D):
    n_chunks = E // CH
    base_iters = n_chunks // NW
    extra = n_chunks % NW
    rows_per_tile = (N // NS) // 8 * 8
    row_rem = N - rows_per_tile * NS
    NB = 3

    @functools.partial(
        pl.kernel,
        out_type=jax.ShapeDtypeStruct((NC, N, D), jnp.float32),
        mesh=_mesh(),
        scratch_types=(
            [pltpu.VMEM_SHARED((N, D), jnp.float32)]
            + [pltpu.VMEM((2, CH), jnp.int32)] * NB
            + [pltpu.VMEM((CH,), jnp.int32)] * NB
            + [pltpu.VMEM((CH, D), jnp.float32)] * NB
            + [pltpu.SemaphoreType.DMA] * (3 * NB)
        ),
    )
    def prop_k(u_hbm, pidx_hbm, part_hbm, acc,
               pidx0, pidx1, pidx2, didx0, didx1, didx2,
               rows0, rows1, rows2,
               isem0, isem1, isem2, gsem0, gsem1, gsem2,
               ssem0, ssem1, ssem2):
        cid = lax.axis_index("c")
        sid = lax.axis_index("s")
        w = sid * NC + cid

        pidx = (pidx0, pidx1, pidx2)
        didx = (didx0, didx1, didx2)
        rows = (rows0, rows1, rows2)
        isem = (isem0, isem1, isem2)
        gsem = (gsem0, gsem1, gsem2)
        ssem = (ssem0, ssem1, ssem2)

        c0 = w * base_iters + jnp.minimum(w, extra)

        for r in range(NB):
            pltpu.async_copy(pidx_hbm.at[c0 + r], pidx[r], isem[r])

        z16 = jnp.zeros((16,), jnp.float32)

        def zfill(r, carry):
            for c in range(D // 16):
                rows0[r, pl.ds(c * 16, 16)] = z16
            return carry
        lax.fori_loop(0, CH, zfill, 0)

        r0 = sid * rows_per_tile
        nfull, ntail = divmod(rows_per_tile, CH)
        for k in range(nfull):
            pltpu.sync_copy(rows0, acc.at[pl.ds(r0 + k * CH, CH)])
        pltpu.sync_copy(rows0.at[pl.ds(0, ntail)],
                        acc.at[pl.ds(r0 + nfull * CH, ntail)])

        @pl.when(sid == 0)
        def _():
            pltpu.sync_copy(rows0.at[pl.ds(0, row_rem)],
                            acc.at[pl.ds(NS * rows_per_tile, row_rem)])

        plsc.subcore_barrier()

        def start_idx(r, b):
            pltpu.async_copy(pidx_hbm.at[c0 + r], pidx[b], isem[b])

        def start_gather(b):
            pltpu.async_copy(u_hbm.at[pidx[b].at[0]], rows[b], gsem[b])

        def wait_idx(b):
            pltpu.make_async_copy(pidx_hbm.at[c0], pidx[b], isem[b]).wait()

        def wait_gather(b):
            pltpu.make_async_copy(u_hbm.at[pidx[b].at[0]], rows[b],
                                  gsem[b]).wait()

        def wait_scatter(b):
            pltpu.make_async_copy(rows[b], acc.at[didx[b]], ssem[b]).wait()

        def scatter(b):
            pltpu.async_copy(rows[b], acc.at[didx[b]], ssem[b], add=True)

        def copy_didx(b):
            for i in range(CH // 16):
                didx[b][pl.ds(i * 16, 16)] = pidx[b][1, pl.ds(i * 16, 16)]

        wait_idx(0)
        start_gather(0)
        wait_idx(1)
        start_gather(1)

        def body(r, b, wait_s, gather_next, idx_next):
            b1 = (b + 2) % NB
            copy_didx(b)
            wait_gather(b)
            if wait_s:
                wait_scatter(b1)
            scatter(b)
            if gather_next:
                wait_idx(b1)
                start_gather(b1)
            if idx_next:
                start_idx(r + NB, b)

        body(0, 0, False, True, True)

        def loop_body(i, carry):
            r = 1 + 3 * i
            body(r, 1, True, True, True)
            body(r + 1, 2, True, True, True)
            body(r + 2, 0, True, True, True)
            return carry
        n_loop = (base_iters - 6) // 3
        lax.fori_loop(0, n_loop, loop_body, 0)

        r = 1 + 3 * n_loop
        body(r, 1, True, True, True)
        body(r + 1, 2, True, True, True)
        body(r + 2, 0, True, True, False)
        body(r + 3, 1, True, False, False)
        body(r + 4, 2, True, False, False)
        wait_scatter(2)

        @pl.when(w < extra)
        def _():
            pltpu.sync_copy(pidx_hbm.at[c0 + base_iters], pidx0)
            pltpu.async_copy(u_hbm.at[pidx0.at[0]], rows0, gsem0).wait()
            pltpu.sync_copy(rows0, acc.at[pidx0.at[1]], add=True)

        plsc.subcore_barrier()

        def wb_start(k, nr, b):
            pltpu.sync_copy(acc.at[pl.ds(r0 + k * CH, nr)],
                            rows[b].at[pl.ds(0, nr)])
            pltpu.async_copy(rows[b].at[pl.ds(0, nr)],
                             part_hbm.at[cid, pl.ds(r0 + k * CH, nr)],
                             gsem[b])

        def wb_wait(nr, b):
            pltpu.make_async_copy(rows[b].at[pl.ds(0, nr)],
                                  part_hbm.at[cid, pl.ds(r0, nr)],
                                  gsem[b]).wait()

        wb_start(0, CH, 0)
        wb_start(1, CH, 1)
        wb_start(2, CH, 2)
        wb_wait(CH, 0)
        wb_start(3, CH, 0)
        wb_wait(CH, 1)
        wb_start(4, ntail, 1)
        wb_wait(CH, 2)
        wb_wait(CH, 0)
        wb_wait(ntail, 1)

        @pl.when(sid == 0)
        def _():
            tail0 = NS * rows_per_tile
            pltpu.sync_copy(acc.at[pl.ds(tail0, row_rem)],
                            rows2.at[pl.ds(0, row_rem)])
            pltpu.sync_copy(rows2.at[pl.ds(0, row_rem)],
                            part_hbm.at[cid, pl.ds(tail0, row_rem)])

    return prop_k


def _dinv_u0_body(degT_ref, emb_ref, dinv_ref, u0_ref):
    deg = degT_ref[:, 0:1] + degT_ref[:, 1:2]
    dinv = jnp.where(deg > 0, 1.0 / jnp.sqrt(deg), 0.0)
    dinv_ref[...] = dinv
    u0_ref[...] = emb_ref[...] * dinv


def _x1_u1_body(part1_ref, dinv_ref, x1_ref, u1_ref):
    dinv = dinv_ref[...]
    x1 = (part1_ref[0] + part1_ref[1]) * dinv
    x1_ref[...] = x1
    u1_ref[...] = x1 * dinv


def _final_body(emb_ref, x1_ref, part2_ref, dinv_ref,
                W1_ref, b1_ref, W2_ref, b2_ref, A1w_ref, A1b_ref,
                A2t_ref, A2b_ref, out_ref):
    f32 = jnp.float32
    emb = emb_ref[...]
    dinv = dinv_ref[...]
    x2 = (part2_ref[0] + part2_ref[1]) * dinv
    zp = (emb + x1_ref[...] + x2) * (1.0 / 3.0)
    h = jnp.maximum(
        jnp.dot(emb, W1_ref[...], preferred_element_type=f32) + b1_ref[...],
        0.0)
    zdp = jnp.maximum(
        jnp.dot(h, W2_ref[...], preferred_element_type=f32) + b2_ref[...],
        0.0)
    t1 = jnp.tanh(
        jnp.dot(zp, A1w_ref[...], preferred_element_type=f32) + A1b_ref[...])
    t2 = jnp.tanh(
        jnp.dot(zdp, A1w_ref[...], preferred_element_type=f32) + A1b_ref[...])
    c0 = jnp.sum(t1 * A2t_ref[0:1, :], axis=1, keepdims=True) + A2b_ref[:, 0:1]
    c1 = jnp.sum(t1 * A2t_ref[1:2, :], axis=1, keepdims=True) + A2b_ref[:, 1:2]
    c2 = jnp.sum(t2 * A2t_ref[0:1, :], axis=1, keepdims=True) + A2b_ref[:, 0:1]
    c3 = jnp.sum(t2 * A2t_ref[1:2, :], axis=1, keepdims=True) + A2b_ref[:, 1:2]
    m = jnp.maximum(jnp.maximum(c0, c1), jnp.maximum(c2, c3))
    e0 = jnp.exp(c0 - m)
    e1 = jnp.exp(c1 - m)
    e2 = jnp.exp(c2 - m)
    e3 = jnp.exp(c3 - m)
    zsum = e0 + e1 + e2 + e3
    out_ref[...] = (e0 / zsum) * zp + (e1 / zsum) * zdp


_BN = 2000


def _row_spec(width):
    return pl.BlockSpec((_BN, width), lambda i: (i, 0))


def _full_spec(shape):
    return pl.BlockSpec(shape, lambda i: (0, 0))


def kernel(edge_index, edge_label_index, emb, W1, b1, W2, b2,
           A1w, A1b, A2w, A2b):
    del edge_label_index
    N, D = emb.shape
    E = edge_index.shape[1]
    f32 = jnp.float32
    grid = (N // _BN,)

    ei = edge_index.astype(jnp.int32)

    pidx_packed = jnp.stack(
        [ei[0].reshape(E // CH, CH), ei[1].reshape(E // CH, CH)], axis=1)

    A2t = A2w.T
    A2br = A2b.reshape(1, 2)

    dst3 = ei[1].reshape(E // CH, 1, CH)
    degT = _deg_kernel(E, N)(dst3).reshape(NC, N).T

    dinv, u0 = pl.pallas_call(
        _dinv_u0_body,
        grid=grid,
        in_specs=[_row_spec(2), _row_spec(D)],
        out_specs=[_row_spec(1), _row_spec(D)],
        out_shape=[jax.ShapeDtypeStruct((N, 1), f32),
                   jax.ShapeDtypeStruct((N, D), f32)],
    )(degT, emb)

    part1 = _prop_kernel(E, N, D)(u0, pidx_packed)

    x1, u1 = pl.pallas_call(
        _x1_u1_body,
        grid=grid,
        in_specs=[pl.BlockSpec((NC, _BN, D), lambda i: (0, i, 0)),
                  _row_spec(1)],
        out_specs=[_row_spec(D), _row_spec(D)],
        out_shape=[jax.ShapeDtypeStruct((N, D), f32),
                   jax.ShapeDtypeStruct((N, D), f32)],
    )(part1, dinv)

    part2 = _prop_kernel(E, N, D)(u1, pidx_packed)

    z = pl.pallas_call(
        _final_body,
        grid=grid,
        in_specs=[
            _row_spec(D), _row_spec(D),
            pl.BlockSpec((NC, _BN, D), lambda i: (0, i, 0)),
            _row_spec(1),
            _full_spec((D, D)), _full_spec((1, D)),
            _full_spec((D, D)), _full_spec((1, D)),
            _full_spec((D, D)), _full_spec((1, D)),
            _full_spec((2, D)), _full_spec((1, 2)),
        ],
        out_specs=_row_spec(D),
        out_shape=jax.ShapeDtypeStruct((N, D), f32),
    )(emb, x1, part2, dinv,
      W1, b1.reshape(1, D), W2, b2.reshape(1, D),
      A1w, A1b.reshape(1, D), A2t, A2br)
    return z


# speedup vs baseline: 1.0254x; 1.0254x over previous
import functools

import jax
import jax.numpy as jnp
from jax import lax
from jax.experimental import pallas as pl
from jax.experimental.pallas import tpu as pltpu
from jax.experimental.pallas import tpu_sc as plsc

NC = 2
NS = 16
NW = NC * NS
CH = 128


def _mesh():
    return plsc.VectorSubcoreMesh(
        core_axis_name="c", subcore_axis_name="s",
        num_cores=NC, num_subcores=NS)


@functools.lru_cache(maxsize=None)
def _deg_kernel(E, N):
    n_chunks = E // CH
    base_iters = n_chunks // NW
    extra = n_chunks % NW
    zlen = (N // NS) // 8 * 8
    rem = N - zlen * NS

    @functools.partial(
        pl.kernel,
        out_type=jax.ShapeDtypeStruct((NC * N,), jnp.float32),
        mesh=_mesh(),
        scratch_types=[
            pltpu.VMEM_SHARED((N,), jnp.float32),
            pltpu.VMEM((2, CH), jnp.int32),
            pltpu.VMEM((2, CH), jnp.int32),
            pltpu.VMEM((CH,), jnp.int32),
            pltpu.VMEM((CH,), jnp.int32),
            pltpu.VMEM((CH,), jnp.float32),
            pltpu.VMEM((zlen,), jnp.float32),
            pltpu.SemaphoreType.DMA,
            pltpu.SemaphoreType.DMA,
            pltpu.SemaphoreType.DMA,
            pltpu.SemaphoreType.DMA,
        ],
    )
    def deg_k(pidx_hbm, degp_hbm, acc, pidx0, pidx1, didx0, didx1,
              ones, zbuf, isem0, isem1, ssem0, ssem1):
        cid = lax.axis_index("c")
        sid = lax.axis_index("s")
        w = sid * NC + cid

        c0 = w * base_iters + jnp.minimum(w, extra)
        pidx = (pidx0, pidx1)
        didx = (didx0, didx1)
        isem = (isem0, isem1)
        ssem = (ssem0, ssem1)

        pltpu.async_copy(pidx_hbm.at[c0], pidx0, isem0)
        pltpu.async_copy(pidx_hbm.at[c0 + 1], pidx1, isem1)

        one16 = jnp.full((16,), 1.0, jnp.float32)
        for i in range(CH // 16):
            ones[pl.ds(i * 16, 16)] = one16
        z16 = jnp.zeros((16,), jnp.float32)

        def zfill(r, carry):
            zbuf[pl.ds(r * 16, 16)] = z16
            return carry
        lax.fori_loop(0, zlen // 16, zfill, 0)

        pltpu.sync_copy(zbuf, acc.at[pl.ds(sid * zlen, zlen)])

        @pl.when(sid == 0)
        def _():
            pltpu.sync_copy(zbuf.at[pl.ds(0, rem)],
                            acc.at[pl.ds(NS * zlen, rem)])

        plsc.subcore_barrier()

        def wait_scatter(b):
            pltpu.make_async_copy(ones, acc.at[didx[b]], ssem[b]).wait()

        def body(r, b, idx_next, wait_s):
            pltpu.make_async_copy(pidx_hbm.at[c0], pidx[b],
                                  isem[b]).wait()
            for i in range(CH // 16):
                didx[b][pl.ds(i * 16, 16)] = pidx[b][1, pl.ds(i * 16, 16)]
            if wait_s:
                wait_scatter(1 - b)
            pltpu.async_copy(ones, acc.at[didx[b]], ssem[b], add=True)
            if idx_next:
                pltpu.async_copy(pidx_hbm.at[c0 + r + 2], pidx[b], isem[b])

        body(0, 0, True, False)

        def loop_body(i, carry):
            r = 1 + 2 * i
            body(r, 1, True, True)
            body(r + 1, 0, True, True)
            return carry
        lax.fori_loop(0, (base_iters - 4) // 2, loop_body, 0)

        body(base_iters - 3, 1, True, True)
        body(base_iters - 2, 0, False, True)
        body(base_iters - 1, 1, False, True)
        wait_scatter(1)

        @pl.when(w < extra)
        def _():
            pltpu.sync_copy(pidx_hbm.at[c0 + base_iters], pidx0)
            pltpu.sync_copy(ones, acc.at[pidx0.at[1]], add=True)

        plsc.subcore_barrier()

        pltpu.sync_copy(acc.at[pl.ds(sid * zlen, zlen)], zbuf)
        pltpu.sync_copy(zbuf, degp_hbm.at[pl.ds(cid * N + sid * zlen, zlen)])

        @pl.when(sid == 0)
        def _():
            pltpu.sync_copy(acc.at[pl.ds(NS * zlen, rem)],
                            zbuf.at[pl.ds(0, rem)])
            pltpu.sync_copy(zbuf.at[pl.ds(0, rem)],
                            degp_hbm.at[pl.ds(cid * N + NS * zlen, rem)])

    return deg_k


@functools.lru_cache(maxsize=None)
def _prop_kernel(E, N, D):
    n_chunks = E // CH
    base_iters = n_chunks // NW
    extra = n_chunks % NW
    rows_per_tile = (N // NS) // 8 * 8
    row_rem = N - rows_per_tile * NS
    NB = 3

    @functools.partial(
        pl.kernel,
        out_type=jax.ShapeDtypeStruct((NC, N, D), jnp.float32),
        mesh=_mesh(),
        scratch_types=(
            [pltpu.VMEM_SHARED((N, D), jnp.float32)]
            + [pltpu.VMEM((2, CH), jnp.int32)] * NB
            + [pltpu.VMEM((CH,), jnp.int32)] * NB
            + [pltpu.VMEM((CH, D), jnp.float32)] * NB
            + [pltpu.SemaphoreType.DMA] * (3 * NB)
        ),
    )
    def prop_k(u_hbm, pidx_hbm, part_hbm, acc,
               pidx0, pidx1, pidx2, didx0, didx1, didx2,
               rows0, rows1, rows2,
               isem0, isem1, isem2, gsem0, gsem1, gsem2,
               ssem0, ssem1, ssem2):
        cid = lax.axis_index("c")
        sid = lax.axis_index("s")
        w = sid * NC + cid

        pidx = (pidx0, pidx1, pidx2)
        didx = (didx0, didx1, didx2)
        rows = (rows0, rows1, rows2)
        isem = (isem0, isem1, isem2)
        gsem = (gsem0, gsem1, gsem2)
        ssem = (ssem0, ssem1, ssem2)

        c0 = w * base_iters + jnp.minimum(w, extra)

        for r in range(NB):
            pltpu.async_copy(pidx_hbm.at[c0 + r], pidx[r], isem[r])

        z16 = jnp.zeros((16,), jnp.float32)

        def zfill(r, carry):
            for c in range(D // 16):
                rows0[r, pl.ds(c * 16, 16)] = z16
            return carry
        lax.fori_loop(0, CH, zfill, 0)

        r0 = sid * rows_per_tile
        nfull, ntail = divmod(rows_per_tile, CH)
        for k in range(nfull):
            pltpu.sync_copy(rows0, acc.at[pl.ds(r0 + k * CH, CH)])
        pltpu.sync_copy(rows0.at[pl.ds(0, ntail)],
                        acc.at[pl.ds(r0 + nfull * CH, ntail)])

        @pl.when(sid == 0)
        def _():
            pltpu.sync_copy(rows0.at[pl.ds(0, row_rem)],
                            acc.at[pl.ds(NS * rows_per_tile, row_rem)])

        plsc.subcore_barrier()

        def start_idx(r, b):
            pltpu.async_copy(pidx_hbm.at[c0 + r], pidx[b], isem[b])

        def start_gather(b):
            pltpu.async_copy(u_hbm.at[pidx[b].at[0]], rows[b], gsem[b])

        def wait_idx(b):
            pltpu.make_async_copy(pidx_hbm.at[c0], pidx[b], isem[b]).wait()

        def wait_gather(b):
            pltpu.make_async_copy(u_hbm.at[pidx[b].at[0]], rows[b],
                                  gsem[b]).wait()

        def wait_scatter(b):
            pltpu.make_async_copy(rows[b], acc.at[didx[b]], ssem[b]).wait()

        def scatter(b):
            pltpu.async_copy(rows[b], acc.at[didx[b]], ssem[b], add=True)

        def copy_didx(b):
            for i in range(CH // 16):
                didx[b][pl.ds(i * 16, 16)] = pidx[b][1, pl.ds(i * 16, 16)]

        wait_idx(0)
        start_gather(0)
        wait_idx(1)
        start_gather(1)

        def body(r, b, wait_s, gather_next, idx_next):
            b1 = (b + 2) % NB
            copy_didx(b)
            wait_gather(b)
            if wait_s:
                wait_scatter(b1)
            scatter(b)
            if gather_next:
                wait_idx(b1)
                start_gather(b1)
            if idx_next:
                start_idx(r + NB, b)

        body(0, 0, False, True, True)

        def loop_body(i, carry):
            r = 1 + 3 * i
            body(r, 1, True, True, True)
            body(r + 1, 2, True, True, True)
            body(r + 2, 0, True, True, True)
            return carry
        n_loop = (base_iters - 6) // 3
        lax.fori_loop(0, n_loop, loop_body, 0)

        r = 1 + 3 * n_loop
        body(r, 1, True, True, True)
        body(r + 1, 2, True, True, True)
        body(r + 2, 0, True, True, False)
        body(r + 3, 1, True, False, False)
        body(r + 4, 2, True, False, False)
        wait_scatter(2)

        @pl.when(w < extra)
        def _():
            pltpu.sync_copy(pidx_hbm.at[c0 + base_iters], pidx0)
            pltpu.async_copy(u_hbm.at[pidx0.at[0]], rows0, gsem0).wait()
            pltpu.sync_copy(rows0, acc.at[pidx0.at[1]], add=True)

        plsc.subcore_barrier()

        def wb_start(k, nr, b):
            pltpu.sync_copy(acc.at[pl.ds(r0 + k * CH, nr)],
                            rows[b].at[pl.ds(0, nr)])
            pltpu.async_copy(rows[b].at[pl.ds(0, nr)],
                             part_hbm.at[cid, pl.ds(r0 + k * CH, nr)],
                             gsem[b])

        def wb_wait(nr, b):
            pltpu.make_async_copy(rows[b].at[pl.ds(0, nr)],
                                  part_hbm.at[cid, pl.ds(r0, nr)],
                                  gsem[b]).wait()

        wb_start(0, CH, 0)
        wb_start(1, CH, 1)
        wb_start(2, CH, 2)
        wb_wait(CH, 0)
        wb_start(3, CH, 0)
        wb_wait(CH, 1)
        wb_start(4, ntail, 1)
        wb_wait(CH, 2)
        wb_wait(CH, 0)
        wb_wait(ntail, 1)

        @pl.when(sid == 0)
        def _():
            tail0 = NS * rows_per_tile
            pltpu.sync_copy(acc.at[pl.ds(tail0, row_rem)],
                            rows2.at[pl.ds(0, row_rem)])
            pltpu.sync_copy(rows2.at[pl.ds(0, row_rem)],
                            part_hbm.at[cid, pl.ds(tail0, row_rem)])

    return prop_k


def _dinv_u0_body(degT_ref, emb_ref, dinv_ref, u0_ref):
    deg = degT_ref[:, 0:1] + degT_ref[:, 1:2]
    dinv = jnp.where(deg > 0, 1.0 / jnp.sqrt(deg), 0.0)
    dinv_ref[...] = dinv
    u0_ref[...] = emb_ref[...] * dinv


def _x1_u1_body(part1_ref, dinv_ref, x1_ref, u1_ref):
    dinv = dinv_ref[...]
    x1 = (part1_ref[0] + part1_ref[1]) * dinv
    x1_ref[...] = x1
    u1_ref[...] = x1 * dinv


def _final_body(emb_ref, x1_ref, part2_ref, dinv_ref,
                W1_ref, b1_ref, W2_ref, b2_ref, A1w_ref, A1b_ref,
                A2t_ref, A2b_ref, out_ref):
    f32 = jnp.float32
    emb = emb_ref[...]
    dinv = dinv_ref[...]
    x2 = (part2_ref[0] + part2_ref[1]) * dinv
    zp = (emb + x1_ref[...] + x2) * (1.0 / 3.0)
    h = jnp.maximum(
        jnp.dot(emb, W1_ref[...], preferred_element_type=f32) + b1_ref[...],
        0.0)
    zdp = jnp.maximum(
        jnp.dot(h, W2_ref[...], preferred_element_type=f32) + b2_ref[...],
        0.0)
    t1 = jnp.tanh(
        jnp.dot(zp, A1w_ref[...], preferred_element_type=f32) + A1b_ref[...])
    t2 = jnp.tanh(
        jnp.dot(zdp, A1w_ref[...], preferred_element_type=f32) + A1b_ref[...])
    c0 = jnp.sum(t1 * A2t_ref[0:1, :], axis=1, keepdims=True) + A2b_ref[:, 0:1]
    c1 = jnp.sum(t1 * A2t_ref[1:2, :], axis=1, keepdims=True) + A2b_ref[:, 1:2]
    c2 = jnp.sum(t2 * A2t_ref[0:1, :], axis=1, keepdims=True) + A2b_ref[:, 0:1]
    c3 = jnp.sum(t2 * A2t_ref[1:2, :], axis=1, keepdims=True) + A2b_ref[:, 1:2]
    m = jnp.maximum(jnp.maximum(c0, c1), jnp.maximum(c2, c3))
    e0 = jnp.exp(c0 - m)
    e1 = jnp.exp(c1 - m)
    e2 = jnp.exp(c2 - m)
    e3 = jnp.exp(c3 - m)
    zsum = e0 + e1 + e2 + e3
    out_ref[...] = (e0 / zsum) * zp + (e1 / zsum) * zdp


_BN = 2000


def _row_spec(width):
    return pl.BlockSpec((_BN, width), lambda i: (i, 0))


def _full_spec(shape):
    return pl.BlockSpec(shape, lambda i: (0, 0))


def kernel(edge_index, edge_label_index, emb, W1, b1, W2, b2,
           A1w, A1b, A2w, A2b):
    del edge_label_index
    N, D = emb.shape
    E = edge_index.shape[1]
    f32 = jnp.float32
    grid = (N // _BN,)

    ei = edge_index.astype(jnp.int32)

    pidx_packed = jnp.stack(
        [ei[0].reshape(E // CH, CH), ei[1].reshape(E // CH, CH)], axis=1)

    A2t = A2w.T
    A2br = A2b.reshape(1, 2)

    degT = _deg_kernel(E, N)(pidx_packed).reshape(NC, N).T

    dinv, u0 = pl.pallas_call(
        _dinv_u0_body,
        grid=grid,
        in_specs=[_row_spec(2), _row_spec(D)],
        out_specs=[_row_spec(1), _row_spec(D)],
        out_shape=[jax.ShapeDtypeStruct((N, 1), f32),
                   jax.ShapeDtypeStruct((N, D), f32)],
    )(degT, emb)

    part1 = _prop_kernel(E, N, D)(u0, pidx_packed)

    x1, u1 = pl.pallas_call(
        _x1_u1_body,
        grid=grid,
        in_specs=[pl.BlockSpec((NC, _BN, D), lambda i: (0, i, 0)),
                  _row_spec(1)],
        out_specs=[_row_spec(D), _row_spec(D)],
        out_shape=[jax.ShapeDtypeStruct((N, D), f32),
                   jax.ShapeDtypeStruct((N, D), f32)],
    )(part1, dinv)

    part2 = _prop_kernel(E, N, D)(u1, pidx_packed)

    z = pl.pallas_call(
        _final_body,
        grid=grid,
        in_specs=[
            _row_spec(D), _row_spec(D),
            pl.BlockSpec((NC, _BN, D), lambda i: (0, i, 0)),
            _row_spec(1),
            _full_spec((D, D)), _full_spec((1, D)),
            _full_spec((D, D)), _full_spec((1, D)),
            _full_spec((D, D)), _full_spec((1, D)),
            _full_spec((2, D)), _full_spec((1, 2)),
        ],
        out_specs=_row_spec(D),
        out_shape=jax.ShapeDtypeStruct((N, D), f32),
    )(emb, x1, part2, dinv,
      W1, b1.reshape(1, D), W2, b2.reshape(1, D),
      A1w, A1b.reshape(1, D), A2t, A2br)
    return z
